# baseline (speedup 1.0000x reference)
import functools

import jax
import jax.numpy as jnp
from jax import lax
from jax.experimental import pallas as pl
from jax.experimental.pallas import tpu as pltpu
from jax.experimental.pallas import tpu_sc as plsc

_NC = 2
_NS = 16
_NW = _NC * _NS
_LANES = 16

_CHUNK = 4096


def _fold_coefs(a2, b2, basis, omega):
  n_rows, d = basis.shape[0], omega.shape[1]
  n_poly = n_rows // 4

  def body(a_ref, b_ref, basis_ref, omega_ref, o_ref):
    coefs = jnp.dot(basis_ref[...], omega_ref[...],
                    preferred_element_type=jnp.float32)
    r = lax.broadcasted_iota(jnp.int32, coefs.shape, 0)
    seg = (r // 4).astype(jnp.float32)
    i = r % 4
    av = a_ref[...]
    bav = (b_ref[...] - av) * (1.0 / n_poly)
    adj0 = av + seg * bav
    adj1 = jnp.broadcast_to(bav, coefs.shape)
    zero = jnp.zeros_like(coefs)
    o_ref[...] = coefs + jnp.where(i == 0, adj0,
                                   jnp.where(i == 1, adj1, zero))

  return pl.pallas_call(
      body,
      out_shape=jax.ShapeDtypeStruct((n_rows, d), jnp.float32),
  )(a2, b2, basis, omega)


def _sc_spline(t, cflat, n, d):
  per_w = n // _NW
  n_chunks = per_w // _CHUNK
  steps = _CHUNK // _LANES
  mesh = plsc.VectorSubcoreMesh(
      core_axis_name="c", subcore_axis_name="s",
      num_cores=_NC, num_subcores=_NS)

  @functools.partial(
      pl.kernel,
      out_type=jax.ShapeDtypeStruct((n * d,), jnp.float32),
      mesh=mesh,
      compiler_params=pltpu.CompilerParams(needs_layout_passes=False),
      scratch_types=[
          pltpu.VMEM((4 * 4 * d,), jnp.float32),
          pltpu.VMEM((_CHUNK,), jnp.float32),
          pltpu.VMEM((_CHUNK * d,), jnp.float32),
      ],
  )
  def run(t_hbm, c_hbm, out_hbm, cbuf, tbuf, obuf):
    wid = lax.axis_index("s") * _NC + lax.axis_index("c")
    pltpu.sync_copy(c_hbm, cbuf)
    base_el = wid * per_w
    io8 = lax.iota(jnp.int32, _LANES) * d

    def chunk_body(ci, carry):
      start = base_el + ci * _CHUNK
      pltpu.sync_copy(t_hbm.at[pl.ds(start, _CHUNK)], tbuf)

      @plsc.parallel_loop(0, steps, unroll=4)
      def step(j):
        tv = tbuf[pl.ds(j * _LANES, _LANES)]
        t4 = tv * 4.0
        segi = jnp.minimum(t4.astype(jnp.int32), 3)
        lt = t4 - segi.astype(jnp.float32)
        lt2 = lt * lt
        lt3 = lt2 * lt
        gb = segi * (4 * d)
        sb = io8 + j * (_LANES * d)
        for dd in range(d):
          c0 = plsc.load_gather(cbuf, [gb + dd])
          c1 = plsc.load_gather(cbuf, [gb + (d + dd)])
          c2_ = plsc.load_gather(cbuf, [gb + (2 * d + dd)])
          c3 = plsc.load_gather(cbuf, [gb + (3 * d + dd)])
          po = c0 + c1 * lt + c2_ * lt2 + c3 * lt3
          plsc.store_scatter(obuf, [sb + dd], po)
      pltpu.sync_copy(obuf, out_hbm.at[pl.ds(start * d, _CHUNK * d)])
      return carry

    lax.fori_loop(0, n_chunks, chunk_body, 0)

  return run(t, cflat)


def kernel(t, a, b, basis, omega):
  n = t.shape[0]
  d = omega.shape[1]
  cfold = _fold_coefs(a.reshape(1, d), b.reshape(1, d), basis, omega)
  cflat = cfold.reshape(-1)
  out_flat = _sc_spline(t, cflat, n, d)
  return out_flat.reshape(n, d)


# speedup vs baseline: 20.0391x; 20.0391x over previous
import functools

import jax
import jax.numpy as jnp
from jax import lax
from jax.experimental import pallas as pl
from jax.experimental.pallas import tpu as pltpu
from jax.experimental.pallas import tpu_sc as plsc

_NC = 2
_NS = 16
_NW = _NC * _NS
_LANES = 16

_CHUNK = 4096


def _fold_coefs(a2, b2, basis, omega):
  n_rows, d = basis.shape[0], omega.shape[1]
  n_poly = n_rows // 4

  def body(a_ref, b_ref, basis_ref, omega_ref, o_ref):
    coefs = jnp.dot(basis_ref[...], omega_ref[...],
                    preferred_element_type=jnp.float32)
    r = lax.broadcasted_iota(jnp.int32, coefs.shape, 0)
    seg = (r // 4).astype(jnp.float32)
    i = r % 4
    av = a_ref[...]
    bav = (b_ref[...] - av) * (1.0 / n_poly)
    adj0 = av + seg * bav
    adj1 = jnp.broadcast_to(bav, coefs.shape)
    zero = jnp.zeros_like(coefs)
    o_ref[...] = coefs + jnp.where(i == 0, adj0,
                                   jnp.where(i == 1, adj1, zero))

  return pl.pallas_call(
      body,
      out_shape=jax.ShapeDtypeStruct((n_rows, d), jnp.float32),
  )(a2, b2, basis, omega)


def _sc_spline(t, ctab, n, d):
  per_w = n // _NW
  n_chunks = per_w // _CHUNK
  n_pairs = n_chunks // 2
  steps = _CHUNK // _LANES
  sub_per_blk = 128 // _LANES
  mesh = plsc.VectorSubcoreMesh(
      core_axis_name="c", subcore_axis_name="s",
      num_cores=_NC, num_subcores=_NS)

  @functools.partial(
      pl.kernel,
      out_type=jax.ShapeDtypeStruct((n * d,), jnp.float32),
      mesh=mesh,
      compiler_params=pltpu.CompilerParams(needs_layout_passes=False),
      scratch_types=[
          pltpu.VMEM((4 * d,), jnp.float32),
          pltpu.VMEM((4 * d,), jnp.float32),
          pltpu.VMEM((4 * d,), jnp.float32),
          pltpu.VMEM((4 * d,), jnp.float32),
          pltpu.VMEM((_CHUNK,), jnp.float32),
          pltpu.VMEM((_CHUNK,), jnp.float32),
          pltpu.VMEM((_CHUNK * 8,), jnp.float32),
          pltpu.VMEM((_CHUNK * 8,), jnp.float32),
          pltpu.SemaphoreType.DMA,
          pltpu.SemaphoreType.DMA,
          pltpu.SemaphoreType.DMA,
          pltpu.SemaphoreType.DMA,
      ],
  )
  def run(t_hbm, c_hbm, out_hbm, cb0, cb1, cb2, cb3,
          tba, tbb, oba, obb, tsa, tsb, osa, osb):
    wid = lax.axis_index("s") * _NC + lax.axis_index("c")
    for i, cb in enumerate((cb0, cb1, cb2, cb3)):
      pltpu.sync_copy(c_hbm.at[i], cb)
    base_el = wid * per_w

    def t_slice(ci):
      return t_hbm.at[pl.ds(base_el + ci * _CHUNK, _CHUNK)]

    def o_slice(ci):
      return out_hbm.at[pl.ds((base_el + ci * _CHUNK) * d, _CHUNK * d)]

    def compute(tb, ob):
      @plsc.parallel_loop(0, steps, unroll=2)
      def step(j):
        tv = tb[pl.ds(j * _LANES, _LANES)]
        t4 = tv * 4.0
        segi = jnp.minimum(t4.astype(jnp.int32), 3)
        lt = t4 - segi.astype(jnp.float32)
        lt2 = lt * lt
        lt3 = lt2 * lt
        gi = segi * d
        boff = (j // sub_per_blk) * (128 * d) + (j % sub_per_blk) * _LANES
        for dd in range(d):
          g = gi + dd
          c0 = plsc.load_gather(cb0, [g])
          c1 = plsc.load_gather(cb1, [g])
          c2 = plsc.load_gather(cb2, [g])
          c3 = plsc.load_gather(cb3, [g])
          po = c0 + c1 * lt + c2 * lt2 + c3 * lt3
          ob[pl.ds(boff + dd * 128, _LANES)] = po

    pltpu.async_copy(t_slice(0), tba, tsa)
    pltpu.async_copy(t_slice(1), tbb, tsb)

    def pair_body(k, carry):
      c0i = 2 * k
      c1i = c0i + 1
      pltpu.make_async_copy(t_slice(c0i), tba, tsa).wait()

      @pl.when(k > 0)
      def _():
        pltpu.make_async_copy(oba, o_slice(c0i), osa).wait()

      compute(tba, oba)
      pltpu.async_copy(oba, o_slice(c0i), osa)

      @pl.when(k < n_pairs - 1)
      def _():
        pltpu.async_copy(t_slice(c0i + 2), tba, tsa)

      pltpu.make_async_copy(t_slice(c1i), tbb, tsb).wait()

      @pl.when(k > 0)
      def _():
        pltpu.make_async_copy(obb, o_slice(c1i), osb).wait()

      compute(tbb, obb)
      pltpu.async_copy(obb, o_slice(c1i), osb)

      @pl.when(k < n_pairs - 1)
      def _():
        pltpu.async_copy(t_slice(c1i + 2), tbb, tsb)

      return carry

    lax.fori_loop(0, n_pairs, pair_body, 0)
    pltpu.make_async_copy(oba, o_slice(n_chunks - 2), osa).wait()
    pltpu.make_async_copy(obb, o_slice(n_chunks - 1), osb).wait()

  return run(t, ctab)


def kernel(t, a, b, basis, omega):
  n = t.shape[0]
  d = omega.shape[1]
  cfold = _fold_coefs(a.reshape(1, d), b.reshape(1, d), basis, omega)
  ctab = cfold.reshape(4, 4, d).transpose(1, 0, 2).reshape(4, 4 * d)
  flat = _sc_spline(t, ctab, n, d)
  return flat.reshape(n // 128, d, 128).transpose(0, 2, 1).reshape(n, d)


# speedup vs baseline: 20.3744x; 1.0167x over previous
import functools

import jax
import jax.numpy as jnp
from jax import lax
from jax.experimental import pallas as pl
from jax.experimental.pallas import tpu as pltpu
from jax.experimental.pallas import tpu_sc as plsc

_NC = 2
_NS = 16
_NW = _NC * _NS
_LANES = 16

_CHUNK = 4096


def _fold_coefs(a2, b2, basis, omega):
  n_rows, d = basis.shape[0], omega.shape[1]
  n_poly = n_rows // 4

  def body(a_ref, b_ref, basis_ref, omega_ref, o_ref):
    coefs = jnp.dot(basis_ref[...], omega_ref[...],
                    preferred_element_type=jnp.float32)
    r = lax.broadcasted_iota(jnp.int32, coefs.shape, 0)
    seg = (r // 4).astype(jnp.float32)
    i = r % 4
    av = a_ref[...]
    bav = (b_ref[...] - av) * (1.0 / n_poly)
    adj0 = av + seg * bav
    adj1 = jnp.broadcast_to(bav, coefs.shape)
    zero = jnp.zeros_like(coefs)
    o_ref[...] = coefs + jnp.where(i == 0, adj0,
                                   jnp.where(i == 1, adj1, zero))

  return pl.pallas_call(
      body,
      out_shape=jax.ShapeDtypeStruct((n_rows, d), jnp.float32),
  )(a2, b2, basis, omega)


def _sc_spline(t, ctab, n, d):
  per_w = n // _NW
  n_chunks = per_w // _CHUNK
  n_pairs = n_chunks // 2
  steps = _CHUNK // _LANES
  sub_per_blk = 128 // _LANES
  mesh = plsc.VectorSubcoreMesh(
      core_axis_name="c", subcore_axis_name="s",
      num_cores=_NC, num_subcores=_NS)

  @functools.partial(
      pl.kernel,
      out_type=jax.ShapeDtypeStruct((n * d,), jnp.float32),
      mesh=mesh,
      compiler_params=pltpu.CompilerParams(needs_layout_passes=False),
      scratch_types=[
          pltpu.VMEM((16 * d,), jnp.float32),
          pltpu.VMEM((4 * d,), jnp.float32),
          pltpu.VMEM((4 * d,), jnp.float32),
          pltpu.VMEM((4 * d,), jnp.float32),
          pltpu.VMEM((4 * d,), jnp.float32),
          pltpu.VMEM((_CHUNK,), jnp.float32),
          pltpu.VMEM((_CHUNK,), jnp.float32),
          pltpu.VMEM((_CHUNK * 8,), jnp.float32),
          pltpu.VMEM((_CHUNK * 8,), jnp.float32),
          pltpu.SemaphoreType.DMA,
          pltpu.SemaphoreType.DMA,
          pltpu.SemaphoreType.DMA,
          pltpu.SemaphoreType.DMA,
      ],
  )
  def run(t_hbm, c_hbm, out_hbm, cfb, cb0, cb1, cb2, cb3,
          tba, tbb, oba, obb, tsa, tsb, osa, osb):
    wid = lax.axis_index("s") * _NC + lax.axis_index("c")
    pltpu.sync_copy(c_hbm, cfb)
    io = lax.iota(jnp.int32, _LANES)
    base_pat = (io // d) * (4 * d) + (io % d)
    for i, cb in enumerate((cb0, cb1, cb2, cb3)):
      for h in range(4 * d // _LANES):
        v = plsc.load_gather(
            cfb, [base_pat + (h * (_LANES // d) * (4 * d) + i * d)])
        cb[pl.ds(h * _LANES, _LANES)] = v
    base_el = wid * per_w

    def t_slice(ci):
      return t_hbm.at[pl.ds(base_el + ci * _CHUNK, _CHUNK)]

    def o_slice(ci):
      return out_hbm.at[pl.ds((base_el + ci * _CHUNK) * d, _CHUNK * d)]

    def compute(tb, ob):
      @plsc.parallel_loop(0, steps, unroll=2)
      def step(j):
        tv = tb[pl.ds(j * _LANES, _LANES)]
        t4 = tv * 4.0
        segi = jnp.minimum(t4.astype(jnp.int32), 3)
        lt = t4 - segi.astype(jnp.float32)
        lt2 = lt * lt
        lt3 = lt2 * lt
        gi = segi * d
        boff = (j // sub_per_blk) * (128 * d) + (j % sub_per_blk) * _LANES
        for dd in range(d):
          g = gi + dd
          c0 = plsc.load_gather(cb0, [g])
          c1 = plsc.load_gather(cb1, [g])
          c2 = plsc.load_gather(cb2, [g])
          c3 = plsc.load_gather(cb3, [g])
          po = c0 + c1 * lt + c2 * lt2 + c3 * lt3
          ob[pl.ds(boff + dd * 128, _LANES)] = po

    pltpu.async_copy(t_slice(0), tba, tsa)
    pltpu.async_copy(t_slice(1), tbb, tsb)

    def pair_body(k, carry):
      c0i = 2 * k
      c1i = c0i + 1
      pltpu.make_async_copy(t_slice(c0i), tba, tsa).wait()

      @pl.when(k > 0)
      def _():
        pltpu.make_async_copy(oba, o_slice(c0i), osa).wait()

      compute(tba, oba)
      pltpu.async_copy(oba, o_slice(c0i), osa)

      @pl.when(k < n_pairs - 1)
      def _():
        pltpu.async_copy(t_slice(c0i + 2), tba, tsa)

      pltpu.make_async_copy(t_slice(c1i), tbb, tsb).wait()

      @pl.when(k > 0)
      def _():
        pltpu.make_async_copy(obb, o_slice(c1i), osb).wait()

      compute(tbb, obb)
      pltpu.async_copy(obb, o_slice(c1i), osb)

      @pl.when(k < n_pairs - 1)
      def _():
        pltpu.async_copy(t_slice(c1i + 2), tbb, tsb)

      return carry

    lax.fori_loop(0, n_pairs, pair_body, 0)
    pltpu.make_async_copy(oba, o_slice(n_chunks - 2), osa).wait()
    pltpu.make_async_copy(obb, o_slice(n_chunks - 1), osb).wait()

  return run(t, ctab)


def kernel(t, a, b, basis, omega):
  n = t.shape[0]
  d = omega.shape[1]
  cflat = _fold_coefs(a.reshape(1, d), b.reshape(1, d), basis,
                      omega).reshape(-1)
  flat = _sc_spline(t, cflat, n, d)
  return flat.reshape(n // 128, d, 128).transpose(0, 2, 1).reshape(n, d)
